# baseline (speedup 1.0000x reference)
import jax
import jax.numpy as jnp
from jax.experimental import pallas as pl
from jax.experimental.pallas import tpu as pltpu

STRICT_MAXIMA_BONUS = 10.0
EPS = 1e-07


def _make_body(D, H, W, TR, dtype):
    neg = float('-inf')
    L = 8 * W

    def body(xc_ref, top_ref, bot_ref, pert_ref, coords_ref, y_ref):
        i = pl.program_id(1)
        p = [[pert_ref[r, c] for c in range(3)] for r in range(3)]
        xc = xc_ref[0]

        prev_tail = jnp.concatenate(
            [top_ref[0, :, 0], xc[:, :TR - 1, L - W:]], axis=1)
        xu = jnp.concatenate([prev_tail, xc[:, :, :L - W]], axis=2)
        next_head = jnp.concatenate(
            [xc[:, 1:, :W], bot_ref[0, :, 0]], axis=1)
        xd = jnp.concatenate([xc[:, :, W:], next_head], axis=2)

        lio = jax.lax.broadcasted_iota(jnp.int32, (1, 1, L), 2)
        wpos = jnp.bitwise_and(lio, W - 1)
        is_w0 = wpos == 0
        is_wend = wpos == W - 1

        def lsh(a):
            al = jnp.concatenate([a[:, :, :1], a[:, :, :-1]], axis=2)
            ar = jnp.concatenate([a[:, :, 1:], a[:, :, -1:]], axis=2)
            return (jnp.where(is_w0, a, al),
                    jnp.where(is_wend, a, ar))

        def dsh(a):
            return (jnp.concatenate([a[:1], a[:-1]], axis=0),
                    jnp.concatenate([a[1:], a[-1:]], axis=0))

        xcl, xcr = lsh(xc)
        xul, xur = lsh(xu)
        xdl, xdr = lsh(xd)

        rio = jax.lax.broadcasted_iota(jnp.int32, (1, TR, 1), 1)
        hio = 8 * (i * TR + rio) + lio // W
        xu_m = jnp.where(hio >= 1, xu, neg)
        xd_m = jnp.where(hio <= H - 2, xd, neg)
        xul_m, xur_m = lsh(xu_m)
        xdl_m, xdr_m = lsh(xd_m)
        lmax = jnp.where(is_w0, neg,
                         jnp.maximum(jnp.maximum(xul_m, xdl_m), xcl))
        rmax = jnp.where(is_wend, neg,
                         jnp.maximum(jnp.maximum(xur_m, xdr_m), xcr))
        m8 = jnp.maximum(jnp.maximum(lmax, rmax), jnp.maximum(xu_m, xd_m))
        m9 = jnp.maximum(m8, xc)
        negp = jnp.full((1, TR, L), neg, dtype)
        m9u = jnp.concatenate([negp, m9[:-1]], axis=0)
        m9d = jnp.concatenate([m9[1:], negp], axis=0)
        m = jnp.maximum(jnp.maximum(m9u, m9d), m8)
        nms = xc > m

        u1 = xcr - xcl
        uh = xd - xu
        xc_dm, xc_dp = dsh(xc)
        uh_dm, uh_dp = dsh(uh)
        u1_dm, u1_dp = dsh(u1)

        dxg = 0.5 * u1
        dyg = 0.5 * uh
        dsg = 0.5 * (xc_dp - xc_dm)
        dxx = xcr + xcl - 2.0 * xc
        dyy = xd + xu - 2.0 * xc
        dss = xc_dp + xc_dm - 2.0 * xc
        dxy = 0.25 * ((xdr - xdl) - (xur - xul))
        dys = 0.25 * (uh_dp - uh_dm)
        dxs = 0.25 * (u1_dp - u1_dm)

        a11 = dss + p[0][0]; a12 = dys + p[0][1]; a13 = dxs + p[0][2]
        a21 = dys + p[1][0]; a22 = dyy + p[1][1]; a23 = dxy + p[1][2]
        a31 = dxs + p[2][0]; a32 = dxy + p[2][1]; a33 = dss + p[2][2]
        cof11 = a22 * a33 - a23 * a32
        cof12 = a23 * a31 - a21 * a33
        cof13 = a21 * a32 - a22 * a31
        cof21 = a13 * a32 - a12 * a33
        cof22 = a11 * a33 - a13 * a31
        cof23 = a12 * a31 - a11 * a32
        cof31 = a12 * a23 - a13 * a22
        cof32 = a13 * a21 - a11 * a23
        cof33 = a11 * a22 - a12 * a21
        det = a11 * cof11 + a12 * cof12 + a13 * cof13
        rdet = 1.0 / det
        b1, b2, b3 = dsg, dyg, dxg
        s1 = (cof11 * b1 + cof21 * b2 + cof31 * b3) * rdet
        s2 = (cof12 * b1 + cof22 * b2 + cof32 * b3) * rdet
        s3 = (cof13 * b1 + cof23 * b2 + cof33 * b3) * rdet

        mab = jnp.maximum(jnp.maximum(jnp.abs(s1), jnp.abs(s2)), jnp.abs(s3))
        conv = jnp.logical_and(nms, mab < 0.5)
        d1 = jnp.where(conv, -s1, 0.0)
        d2 = jnp.where(conv, -s2, 0.0)
        d3 = jnp.where(conv, -s3, 0.0)
        dy = 0.5 * (b1 * d1 + b2 * d2 + b3 * d3)
        y_ref[0, 0] = xc + dy + STRICT_MAXIMA_BONUS * conv.astype(dtype)

        fdio = jax.lax.broadcasted_iota(jnp.int32, (D, TR, L), 0).astype(dtype)
        fwio = wpos.astype(dtype)
        fhio = hio.astype(dtype)
        coords_ref[0, 0, 0] = fdio + d1
        coords_ref[0, 0, 1] = fhio + d2
        coords_ref[0, 0, 2] = fwio + d3

    return body


def kernel(x):
    B, C, D, H, W = x.shape
    dtype = x.dtype
    TR = 16
    HR = H // 8
    L = 8 * W
    nT = HR // TR
    xq = x.reshape(B, D, HR, L)
    x4 = x.reshape(B, D, H, W)
    TH = 8 * TR
    top = jnp.concatenate(
        [x4[:, :, :1], x4[:, :, TH - 1::TH][:, :, :nT - 1]], axis=2)
    bot = jnp.concatenate([x4[:, :, TH::TH], x4[:, :, H - 1:]], axis=2)
    top = top.reshape(B, D, nT, 1, W)
    bot = bot.reshape(B, D, nT, 1, W)

    pert = jnp.abs(jax.random.uniform(
        jax.random.fold_in(jax.random.key(0), 7), (3, 3),
        dtype=dtype)) * EPS

    body = _make_body(D, H, W, TR, dtype)
    coords, y = pl.pallas_call(
        body,
        grid=(B, nT),
        compiler_params=pltpu.CompilerParams(
            dimension_semantics=("parallel", "parallel")),
        in_specs=[
            pl.BlockSpec((1, D, TR, L), lambda b, i: (b, 0, i, 0)),
            pl.BlockSpec((1, D, 1, 1, W), lambda b, i: (b, 0, i, 0, 0)),
            pl.BlockSpec((1, D, 1, 1, W), lambda b, i: (b, 0, i, 0, 0)),
            pl.BlockSpec(memory_space=pltpu.SMEM),
        ],
        out_specs=[
            pl.BlockSpec((1, 1, 3, D, TR, L), lambda b, i: (b, 0, 0, 0, i, 0)),
            pl.BlockSpec((1, 1, D, TR, L), lambda b, i: (b, 0, 0, i, 0)),
        ],
        out_shape=[
            jax.ShapeDtypeStruct((B, 1, 3, D, HR, L), dtype),
            jax.ShapeDtypeStruct((B, 1, D, HR, L), dtype),
        ],
    )(xq, top, bot, pert)
    return (coords.reshape(B, 1, 3, D, H, W), y.reshape(B, 1, D, H, W))


# speedup vs baseline: 1.6279x; 1.6279x over previous
import jax
import jax.numpy as jnp
from jax.experimental import pallas as pl
from jax.experimental.pallas import tpu as pltpu

STRICT_MAXIMA_BONUS = 10.0
EPS = 1e-07


def _make_body(D, H, W, TH, dtype):
    neg = float('-inf')

    def body(xu_ref, xc_ref, xd_ref, pert_ref, coords_ref, y_ref):
        i = pl.program_id(1)
        p = [[pert_ref[r, c] for c in range(3)] for r in range(3)]
        xu = xu_ref[0]
        xd = xd_ref[0]
        xc = xc_ref[0]

        def lsh(a):
            return (jnp.concatenate([a[:, :, :1], a[:, :, :-1]], axis=2),
                    jnp.concatenate([a[:, :, 1:], a[:, :, -1:]], axis=2))

        def dsh(a):
            return (jnp.concatenate([a[:1], a[:-1]], axis=0),
                    jnp.concatenate([a[1:], a[-1:]], axis=0))

        xcl, xcr = lsh(xc)
        xul, xur = lsh(xu)
        xdl, xdr = lsh(xd)

        hio = i * TH + jax.lax.broadcasted_iota(jnp.int32, (1, TH, 1), 1)
        xu_m = jnp.where(hio >= 1, xu, neg)
        xd_m = jnp.where(hio <= H - 2, xd, neg)
        xul_m, xur_m = lsh(xu_m)
        xdl_m, xdr_m = lsh(xd_m)
        wio = jax.lax.broadcasted_iota(jnp.int32, (1, 1, W), 2)
        lmax = jnp.where(wio >= 1,
                         jnp.maximum(jnp.maximum(xul_m, xdl_m), xcl), neg)
        rmax = jnp.where(wio <= W - 2,
                         jnp.maximum(jnp.maximum(xur_m, xdr_m), xcr), neg)
        m8 = jnp.maximum(jnp.maximum(lmax, rmax), jnp.maximum(xu_m, xd_m))
        m9 = jnp.maximum(m8, xc)
        negp = jnp.full((1, TH, W), neg, dtype)
        m9u = jnp.concatenate([negp, m9[:-1]], axis=0)
        m9d = jnp.concatenate([m9[1:], negp], axis=0)
        m = jnp.maximum(jnp.maximum(m9u, m9d), m8)
        nms = xc > m

        u1 = xcr - xcl
        uh = xd - xu
        xc_dm, xc_dp = dsh(xc)
        uh_dm, uh_dp = dsh(uh)
        u1_dm, u1_dp = dsh(u1)

        dxg = 0.5 * u1
        dyg = 0.5 * uh
        dsg = 0.5 * (xc_dp - xc_dm)
        dxx = xcr + xcl - 2.0 * xc
        dyy = xd + xu - 2.0 * xc
        dss = xc_dp + xc_dm - 2.0 * xc
        dxy = 0.25 * ((xdr - xdl) - (xur - xul))
        dys = 0.25 * (uh_dp - uh_dm)
        dxs = 0.25 * (u1_dp - u1_dm)

        a11 = dss + p[0][0]; a12 = dys + p[0][1]; a13 = dxs + p[0][2]
        a21 = dys + p[1][0]; a22 = dyy + p[1][1]; a23 = dxy + p[1][2]
        a31 = dxs + p[2][0]; a32 = dxy + p[2][1]; a33 = dss + p[2][2]
        cof11 = a22 * a33 - a23 * a32
        cof12 = a23 * a31 - a21 * a33
        cof13 = a21 * a32 - a22 * a31
        cof21 = a13 * a32 - a12 * a33
        cof22 = a11 * a33 - a13 * a31
        cof23 = a12 * a31 - a11 * a32
        cof31 = a12 * a23 - a13 * a22
        cof32 = a13 * a21 - a11 * a23
        cof33 = a11 * a22 - a12 * a21
        det = a11 * cof11 + a12 * cof12 + a13 * cof13
        rdet = 1.0 / det
        b1, b2, b3 = dsg, dyg, dxg
        s1 = (cof11 * b1 + cof21 * b2 + cof31 * b3) * rdet
        s2 = (cof12 * b1 + cof22 * b2 + cof32 * b3) * rdet
        s3 = (cof13 * b1 + cof23 * b2 + cof33 * b3) * rdet

        mab = jnp.maximum(jnp.maximum(jnp.abs(s1), jnp.abs(s2)), jnp.abs(s3))
        conv = jnp.logical_and(nms, mab < 0.5)
        d1 = jnp.where(conv, -s1, 0.0)
        d2 = jnp.where(conv, -s2, 0.0)
        d3 = jnp.where(conv, -s3, 0.0)
        dy = 0.5 * (b1 * d1 + b2 * d2 + b3 * d3)
        y_ref[0, 0] = xc + dy + STRICT_MAXIMA_BONUS * conv.astype(dtype)

        fdio = jax.lax.broadcasted_iota(jnp.int32, (D, TH, W), 0).astype(dtype)
        fwio = jax.lax.broadcasted_iota(jnp.int32, (D, TH, W), 2).astype(dtype)
        fhio = (i * TH
                + jax.lax.broadcasted_iota(jnp.int32, (D, TH, W), 1)
                ).astype(dtype)
        coords_ref[0, 0, 0] = fdio + d1
        coords_ref[0, 0, 1] = fhio + d2
        coords_ref[0, 0, 2] = fwio + d3

    return body


def kernel(x):
    B, C, D, H, W = x.shape
    dtype = x.dtype
    TH = 128
    nT = H // TH
    x4 = x.reshape(B, D, H, W)
    xu = jnp.concatenate([x4[:, :, :1], x4[:, :, :H - 1]], axis=2)
    xd = jnp.concatenate([x4[:, :, 1:], x4[:, :, H - 1:]], axis=2)

    pert = jnp.abs(jax.random.uniform(
        jax.random.fold_in(jax.random.key(0), 7), (3, 3),
        dtype=dtype)) * EPS

    body = _make_body(D, H, W, TH, dtype)
    coords, y = pl.pallas_call(
        body,
        grid=(B, nT),
        compiler_params=pltpu.CompilerParams(
            dimension_semantics=("parallel", "parallel"),
            allow_input_fusion=[True, False, True, False]),
        in_specs=[
            pl.BlockSpec((1, D, TH, W), lambda b, i: (b, 0, i, 0)),
            pl.BlockSpec((1, D, TH, W), lambda b, i: (b, 0, i, 0)),
            pl.BlockSpec((1, D, TH, W), lambda b, i: (b, 0, i, 0)),
            pl.BlockSpec(memory_space=pltpu.SMEM),
        ],
        out_specs=[
            pl.BlockSpec((1, 1, 3, D, TH, W), lambda b, i: (b, 0, 0, 0, i, 0)),
            pl.BlockSpec((1, 1, D, TH, W), lambda b, i: (b, 0, 0, i, 0)),
        ],
        out_shape=[
            jax.ShapeDtypeStruct((B, 1, 3, D, H, W), dtype),
            jax.ShapeDtypeStruct((B, 1, D, H, W), dtype),
        ],
    )(xu, x4, xd, pert)
    return coords, y
